# Initial kernel scaffold; baseline (speedup 1.0000x reference)
import jax
import jax.numpy as jnp
from jax.experimental import pallas as pl


def kernel(x, edge_index, W0, a0, W1, a1, W2, a2):
    raise NotImplementedError("write your pallas kernel here")


# speedup vs baseline: 1.8028x; 1.8028x over previous
import functools

import jax
import jax.numpy as jnp
from jax import lax
from jax.experimental import pallas as pl
from jax.experimental.pallas import tpu as pltpu
from jax.experimental.pallas import tpu_sc as plsc

N = 10000
NP = 10240
D = 128
E = 320000

NC = 2
NS = 16
L = 16
NW = NC * NS

CHUNK = 128
EPW = 10240
E_PAD = NW * EPW
NCHUNK = EPW // CHUNK
IDX_BLK = 8
RPT = NP // NS

ROW_BLK = 1024
GRID = NP // ROW_BLK


def _sc_edge_body(ft_hbm, src_hbm, dst_hbm, a_hbm, z2_hbm, z1_hbm,
                  u_out, den_out,
                  src_idx, dst_idx, rows_s, rows_d, ex_v, a_v,
                  u_sh, den_sh, sem1, sem2):
    c = lax.axis_index("c")
    s = lax.axis_index("s")
    w = c * NS + s

    pltpu.sync_copy(z2_hbm.at[pl.ds(s * RPT, RPT)],
                    u_sh.at[pl.ds(s * RPT, RPT)])
    pltpu.sync_copy(z1_hbm.at[pl.ds(s * RPT, RPT)],
                    den_sh.at[pl.ds(s * RPT, RPT)])
    pltpu.sync_copy(a_hbm, a_v)
    plsc.subcore_barrier()

    def block_body(jb, carry_b):
        pltpu.sync_copy(src_hbm.at[pl.ds(w * NCHUNK + jb * IDX_BLK, IDX_BLK)],
                        src_idx)
        pltpu.sync_copy(dst_hbm.at[pl.ds(w * NCHUNK + jb * IDX_BLK, IDX_BLK)],
                        dst_idx)

        def chunk_body(jj, carry):
            j = jb * IDX_BLK + jj
            g1 = pltpu.async_copy(ft_hbm.at[src_idx.at[jj]], rows_s, sem1)
            g2 = pltpu.async_copy(ft_hbm.at[dst_idx.at[jj]], rows_d, sem2)
            g1.wait()
            g2.wait()
            ebase = w * EPW + j * CHUNK

            def group_body(g, carry_g):
                eidx = g * L + lax.iota(jnp.int32, L)

                def dot_body(d, acc):
                    dsplat = jnp.full((L,), d, jnp.int32)
                    cs = plsc.load_gather(rows_s, [eidx, dsplat])
                    cd = plsc.load_gather(rows_d, [eidx, dsplat])
                    t = cs + cd
                    lr = jnp.maximum(t, 0.2 * t)
                    av = plsc.load_gather(a_v, [dsplat])
                    return acc + av * lr

                acc = lax.fori_loop(0, D, dot_body,
                                    jnp.zeros((L,), jnp.float32))
                ex = jnp.exp(acc)
                ex = jnp.where(ebase + eidx < E, ex, 0.0)
                ex_v[pl.ds(g * L, L)] = ex

                def scale_body(d, carry_d):
                    dsplat = jnp.full((L,), d, jnp.int32)
                    cs = plsc.load_gather(rows_s, [eidx, dsplat])
                    plsc.store_scatter(rows_s, [eidx, dsplat], cs * ex)
                    return carry_d

                lax.fori_loop(0, D, scale_body, 0)
                return carry_g

            lax.fori_loop(0, CHUNK // L, group_body, 0)
            pltpu.sync_copy(rows_s, u_sh.at[dst_idx.at[jj]], add=True)
            pltpu.sync_copy(ex_v, den_sh.at[dst_idx.at[jj]], add=True)
            return carry

        lax.fori_loop(0, IDX_BLK, chunk_body, 0)
        return carry_b

    lax.fori_loop(0, NCHUNK // IDX_BLK, block_body, 0)
    plsc.subcore_barrier()
    pltpu.sync_copy(u_sh.at[pl.ds(s * RPT, RPT)],
                    u_out.at[c, pl.ds(s * RPT, RPT)])
    pltpu.sync_copy(den_sh.at[pl.ds(s * RPT, RPT)],
                    den_out.at[c, pl.ds(s * RPT, RPT)])


_sc_edge = pl.kernel(
    _sc_edge_body,
    out_type=[
        jax.ShapeDtypeStruct((NC, NP, D), jnp.float32),
        jax.ShapeDtypeStruct((NC, NP), jnp.float32),
    ],
    mesh=plsc.VectorSubcoreMesh(core_axis_name="c", subcore_axis_name="s"),
    compiler_params=pltpu.CompilerParams(needs_layout_passes=False),
    scratch_types=[
        pltpu.VMEM((IDX_BLK, CHUNK), jnp.int32),
        pltpu.VMEM((IDX_BLK, CHUNK), jnp.int32),
        pltpu.VMEM((CHUNK, D), jnp.float32),
        pltpu.VMEM((CHUNK, D), jnp.float32),
        pltpu.VMEM((CHUNK,), jnp.float32),
        pltpu.VMEM((D,), jnp.float32),
        pltpu.VMEM_SHARED((NP, D), jnp.float32),
        pltpu.VMEM_SHARED((NP,), jnp.float32),
        pltpu.SemaphoreType.DMA,
        pltpu.SemaphoreType.DMA,
    ],
)


def _mm_body(x_ref, w_ref, o_ref):
    o_ref[...] = jnp.dot(x_ref[...], w_ref[...],
                         preferred_element_type=jnp.float32)


_mm = pl.pallas_call(
    _mm_body,
    grid=(GRID,),
    in_specs=[
        pl.BlockSpec((ROW_BLK, D), lambda i: (i, 0)),
        pl.BlockSpec((D, D), lambda i: (0, 0)),
    ],
    out_specs=pl.BlockSpec((ROW_BLK, D), lambda i: (i, 0)),
    out_shape=jax.ShapeDtypeStruct((NP, D), jnp.float32),
)


def _combine_body(u_ref, den_ref, res_ref, w_ref, h_ref, ft_ref):
    u = u_ref[0] + u_ref[1]
    dsum = den_ref[:, 0:1] + den_ref[:, 1:2]
    dsafe = jnp.where(dsum == 0.0, 1.0, dsum)
    v = u / dsafe + res_ref[...]
    h = jnp.where(v > 0, v, jnp.exp(v) - 1.0)
    h_ref[...] = h
    ft_ref[...] = jnp.dot(h, w_ref[...], preferred_element_type=jnp.float32)


_combine = pl.pallas_call(
    _combine_body,
    grid=(GRID,),
    in_specs=[
        pl.BlockSpec((NC, ROW_BLK, D), lambda i: (0, i, 0)),
        pl.BlockSpec((ROW_BLK, NC), lambda i: (i, 0)),
        pl.BlockSpec((ROW_BLK, D), lambda i: (i, 0)),
        pl.BlockSpec((D, D), lambda i: (0, 0)),
    ],
    out_specs=[
        pl.BlockSpec((ROW_BLK, D), lambda i: (i, 0)),
        pl.BlockSpec((ROW_BLK, D), lambda i: (i, 0)),
    ],
    out_shape=[
        jax.ShapeDtypeStruct((NP, D), jnp.float32),
        jax.ShapeDtypeStruct((NP, D), jnp.float32),
    ],
)


def _final_body(u_ref, den_ref, res_ref, o_ref):
    i = pl.program_id(0)
    u = u_ref[0] + u_ref[1]
    dsum = den_ref[:, 0:1] + den_ref[:, 1:2]
    dsafe = jnp.where(dsum == 0.0, 1.0, dsum)
    v = u / dsafe + res_ref[...]
    h = jnp.where(v > 0, v, jnp.exp(v) - 1.0)
    part = jnp.sum(h, axis=0, keepdims=True) * (1.0 / N)

    @pl.when(i == 0)
    def _():
        o_ref[...] = jnp.zeros_like(o_ref)

    o_ref[...] += part


_final = pl.pallas_call(
    _final_body,
    grid=(GRID,),
    in_specs=[
        pl.BlockSpec((NC, ROW_BLK, D), lambda i: (0, i, 0)),
        pl.BlockSpec((ROW_BLK, NC), lambda i: (i, 0)),
        pl.BlockSpec((ROW_BLK, D), lambda i: (i, 0)),
    ],
    out_specs=pl.BlockSpec((1, D), lambda i: (0, 0)),
    out_shape=jax.ShapeDtypeStruct((1, D), jnp.float32),
)


def kernel(x, edge_index, W0, a0, W1, a1, W2, a2):
    src = edge_index[0].astype(jnp.int32)
    dst = edge_index[1].astype(jnp.int32)
    pad = jnp.zeros((E_PAD - E,), jnp.int32)
    src2d = jnp.concatenate([src, pad]).reshape(NW * NCHUNK, CHUNK)
    dst2d = jnp.concatenate([dst, pad]).reshape(NW * NCHUNK, CHUNK)
    z2 = jnp.zeros((NP, D), jnp.float32)
    z1 = jnp.zeros((NP,), jnp.float32)
    zres = jnp.zeros((NP, D), jnp.float32)
    xp = jnp.concatenate([x, jnp.zeros((NP - N, D), jnp.float32)])

    ft = _mm(xp, W0)
    u, den = _sc_edge(ft, src2d, dst2d, a0.reshape(D), z2, z1)
    h1, ft = _combine(u, den.T, zres, W1)
    u, den = _sc_edge(ft, src2d, dst2d, a1.reshape(D), z2, z1)
    h2, ft = _combine(u, den.T, h1, W2)
    u, den = _sc_edge(ft, src2d, dst2d, a2.reshape(D), z2, z1)
    return _final(u, den.T, h2)
